# baseline (speedup 1.0000x reference)
import functools

import jax
import jax.numpy as jnp
from jax import lax
from jax.experimental import pallas as pl

_NB = 40


def _body(S_ref, Sj_ref, h_ref, mi_ref, mij_ref, J_ref, U_ref, Ui_ref, *, nb, k, c):
    i = pl.program_id(0)
    sj = Sj_ref[0]
    mij = mij_ref[0]
    iota_s = lax.broadcasted_iota(jnp.int32, (nb, k, c), 2)
    onehot = (sj[:, :, None] == iota_s).astype(jnp.float32) * mij[:, :, None]
    Jb = J_ref[...]
    J_ij = (Jb * onehot[:, :, None, :]).sum(-1)
    J_i = J_ij.sum(1)
    h_m = h_ref[...] * mi_ref[0, 0, :][:, None]
    U_i = h_m + J_i
    Ui_ref[...] = U_i
    s = S_ref[0, 0, :]
    iota_c = lax.broadcasted_iota(jnp.int32, (nb, c), 1)
    sel = (s[:, None] == iota_c).astype(jnp.float32)
    contrib = ((U_i - 0.5 * J_i) * sel).sum().reshape(1, 1)

    @pl.when(i == 0)
    def _():
        U_ref[...] = jnp.zeros((1, 1), jnp.float32)

    U_ref[...] += contrib


def kernel(S, h, J, edge_idx, mask_i, mask_ij):
    B, N, K, C, _ = J.shape
    nb = _NB
    g = N // nb
    assert B == 1 and N % nb == 0

    S1 = S[0]
    Sj = jnp.take(S1, edge_idx[0].reshape(-1), axis=0).reshape(g, nb, K)
    S3 = S1.reshape(g, 1, nb)
    mi3 = mask_i[0].reshape(g, 1, nb)
    mij3 = mask_ij[0].reshape(g, nb, K)
    h2 = h[0]
    J4 = J[0]

    body = functools.partial(_body, nb=nb, k=K, c=C)
    U, U_i = pl.pallas_call(
        body,
        grid=(g,),
        in_specs=[
            pl.BlockSpec((1, 1, nb), lambda i: (i, 0, 0)),
            pl.BlockSpec((1, nb, K), lambda i: (i, 0, 0)),
            pl.BlockSpec((nb, C), lambda i: (i, 0)),
            pl.BlockSpec((1, 1, nb), lambda i: (i, 0, 0)),
            pl.BlockSpec((1, nb, K), lambda i: (i, 0, 0)),
            pl.BlockSpec((nb, K, C, C), lambda i: (i, 0, 0, 0)),
        ],
        out_specs=[
            pl.BlockSpec((1, 1), lambda i: (0, 0)),
            pl.BlockSpec((nb, C), lambda i: (i, 0)),
        ],
        out_shape=[
            jax.ShapeDtypeStruct((1, 1), jnp.float32),
            jax.ShapeDtypeStruct((N, C), jnp.float32),
        ],
    )(S3, Sj, h2, mi3, mij3, J4)
    return (U.reshape(1), U_i.reshape(1, N, C))


# speedup vs baseline: 15.1952x; 15.1952x over previous
import functools

import jax
import jax.numpy as jnp
from jax import lax
from jax.experimental import pallas as pl
from jax.experimental.pallas import tpu as pltpu
from jax.experimental.pallas import tpu_sc as plsc

_NC = 2
_NS = 16
_NW = _NC * _NS


_CHUNK = 128
_FIRE = 8


def _sc_gather_body(S_hbm, edge_hbm, out_hbm, idx_v, out_v, sem, *, e_per_w, rows):
    wid = lax.axis_index("s") * _NC + lax.axis_index("c")
    pltpu.sync_copy(edge_hbm.at[wid], idx_v)

    def round_(r, _):
        cps = []
        for t in range(_FIRE):
            j = r * _FIRE + t
            cps.append(pltpu.make_async_copy(
                S_hbm.at[idx_v.at[j]],
                out_v.at[pl.ds(j * _CHUNK, _CHUNK)],
                sem,
            ))
        for cp in cps:
            cp.start()
        for cp in cps:
            cp.wait()
        return 0

    lax.fori_loop(0, rows // _FIRE, round_, 0)
    pltpu.sync_copy(out_v.at[pl.ds(0, e_per_w)],
                    out_hbm.at[pl.ds(wid * e_per_w, e_per_w)])


def _neighbor_states(S_flat, edge_flat):
    e_per_w = 5000
    _, rows, _ = edge_flat.shape
    mesh = plsc.VectorSubcoreMesh(core_axis_name="c", subcore_axis_name="s")
    body = functools.partial(_sc_gather_body, e_per_w=e_per_w, rows=rows)
    return pl.kernel(
        body,
        mesh=mesh,
        out_type=jax.ShapeDtypeStruct((_NW * e_per_w,), jnp.int32),
        scratch_types=[
            pltpu.VMEM((rows, _CHUNK), jnp.int32),
            pltpu.VMEM((rows * _CHUNK,), jnp.int32),
            pltpu.SemaphoreType.DMA,
        ],
    )(S_flat, edge_flat)


def _tc_body(sj_ref, mij_ref, h_ref, mi_ref, S_ref, J_ref, U_ref, Ui_ref, M_ref, *, c, k):
    ci = pl.program_id(0)

    @pl.when(ci == 0)
    def _():
        sj = sj_ref[...]
        mij = mij_ref[...]
        for s in range(c):
            M_ref[s] = (sj == s).astype(jnp.float32) * mij

    part = J_ref[0, 0] * M_ref[0]
    for s in range(1, c):
        part += J_ref[0, s] * M_ref[s]
    ji = part.sum(axis=0, keepdims=True)
    hm = h_ref[pl.ds(ci, 1), :] * mi_ref[...]
    Ui_ref[pl.ds(ci, 1), :] = hm + ji
    sel = (S_ref[...] == ci).astype(jnp.float32)
    contrib = ((hm + 0.5 * ji) * sel).sum().reshape(1, 1)

    @pl.when(ci == 0)
    def _():
        U_ref[...] = jnp.zeros((1, 1), jnp.float32)

    U_ref[...] += contrib


def kernel(S, h, J, edge_idx, mask_i, mask_ij):
    B, N, K, C, _ = J.shape
    assert B == 1

    S_flat = S[0]
    e_per_w = 5000
    pad_w = _FIRE * _CHUNK * ((e_per_w + _FIRE * _CHUNK - 1) // (_FIRE * _CHUNK))
    edge_w = jnp.transpose(edge_idx[0], (1, 0)).reshape(_NW, e_per_w)
    edge_pad = jnp.pad(edge_w, ((0, 0), (0, pad_w - e_per_w)))
    edge_blocks = edge_pad.reshape(_NW, pad_w // _CHUNK, _CHUNK)
    sj = _neighbor_states(S_flat, edge_blocks).reshape(K, N)

    Jt = jnp.transpose(J[0], (2, 3, 1, 0))
    h_cn = jnp.transpose(h[0], (1, 0))
    mij_kn = jnp.transpose(mask_ij[0], (1, 0))

    body = functools.partial(_tc_body, c=C, k=K)
    U, Ui = pl.pallas_call(
        body,
        grid=(C,),
        in_specs=[
            pl.BlockSpec((K, N), lambda i: (0, 0)),
            pl.BlockSpec((K, N), lambda i: (0, 0)),
            pl.BlockSpec((C, N), lambda i: (0, 0)),
            pl.BlockSpec((1, N), lambda i: (0, 0)),
            pl.BlockSpec((1, N), lambda i: (0, 0)),
            pl.BlockSpec((1, C, K, N), lambda i: (i, 0, 0, 0)),
        ],
        out_specs=[
            pl.BlockSpec((1, 1), lambda i: (0, 0)),
            pl.BlockSpec((C, N), lambda i: (0, 0)),
        ],
        out_shape=[
            jax.ShapeDtypeStruct((1, 1), jnp.float32),
            jax.ShapeDtypeStruct((C, N), jnp.float32),
        ],
        scratch_shapes=[pltpu.VMEM((C, K, N), jnp.float32)],
    )(sj, mij_kn, h_cn, mask_i, S, Jt)
    return (U.reshape(1), jnp.transpose(Ui, (1, 0))[None])


# speedup vs baseline: 19.5596x; 1.2872x over previous
import functools

import jax
import jax.numpy as jnp
from jax import lax
from jax.experimental import pallas as pl
from jax.experimental.pallas import tpu as pltpu
from jax.experimental.pallas import tpu_sc as plsc

_NC = 2
_NS = 16
_NW = _NC * _NS


_CHUNK = 128
_FIRE = 8


def _sc_gather_body(S_hbm, edge_hbm, out_hbm, S_sh, idx_v, out_v, sem, *, e_per_w, rows):
    sid = lax.axis_index("s")
    wid = sid * _NC + lax.axis_index("c")

    @pl.when(sid == 0)
    def _():
        pltpu.sync_copy(S_hbm, S_sh)

    cp_idx = pltpu.make_async_copy(edge_hbm.at[wid], idx_v, sem)
    cp_idx.start()
    plsc.subcore_barrier()
    cp_idx.wait()

    def fire(j, _):
        pltpu.make_async_copy(
            S_sh.at[idx_v.at[j]],
            out_v.at[pl.ds(j * _CHUNK, _CHUNK)],
            sem,
        ).start()
        return 0

    def drain(j, _):
        pltpu.make_async_copy(
            S_sh.at[idx_v.at[j]],
            out_v.at[pl.ds(j * _CHUNK, _CHUNK)],
            sem,
        ).wait()
        return 0

    lax.fori_loop(0, rows, fire, 0)
    lax.fori_loop(0, rows, drain, 0)
    pltpu.sync_copy(out_v.at[pl.ds(0, e_per_w)],
                    out_hbm.at[pl.ds(wid * e_per_w, e_per_w)])


def _neighbor_states(S_flat, edge_flat):
    e_per_w = 5000
    _, rows, _ = edge_flat.shape
    mesh = plsc.VectorSubcoreMesh(core_axis_name="c", subcore_axis_name="s")
    body = functools.partial(_sc_gather_body, e_per_w=e_per_w, rows=rows)
    return pl.kernel(
        body,
        mesh=mesh,
        out_type=jax.ShapeDtypeStruct((_NW * e_per_w,), jnp.int32),
        scratch_types=[
            pltpu.VMEM_SHARED((S_flat.shape[0],), jnp.int32),
            pltpu.VMEM((rows, _CHUNK), jnp.int32),
            pltpu.VMEM((rows * _CHUNK,), jnp.int32),
            pltpu.SemaphoreType.DMA,
        ],
    )(S_flat, edge_flat)


def _tc_body(sj_ref, mij_ref, h_ref, mi_ref, S_ref, J_ref, U_ref, Ui_ref, M_ref, *, c, k):
    ci = pl.program_id(0)

    @pl.when(ci == 0)
    def _():
        sj = sj_ref[...]
        mij = mij_ref[...]
        for s in range(c):
            M_ref[s] = (sj == s).astype(jnp.float32) * mij

    part = J_ref[0, 0] * M_ref[0]
    for s in range(1, c):
        part += J_ref[0, s] * M_ref[s]
    ji = part.sum(axis=0, keepdims=True)
    hm = h_ref[pl.ds(ci, 1), :] * mi_ref[...]
    Ui_ref[pl.ds(ci, 1), :] = hm + ji
    sel = (S_ref[...] == ci).astype(jnp.float32)
    contrib = ((hm + 0.5 * ji) * sel).sum().reshape(1, 1)

    @pl.when(ci == 0)
    def _():
        U_ref[...] = jnp.zeros((1, 1), jnp.float32)

    U_ref[...] += contrib


def kernel(S, h, J, edge_idx, mask_i, mask_ij):
    B, N, K, C, _ = J.shape
    assert B == 1

    S_flat = S[0]
    e_per_w = 5000
    pad_w = _FIRE * _CHUNK * ((e_per_w + _FIRE * _CHUNK - 1) // (_FIRE * _CHUNK))
    edge_w = jnp.transpose(edge_idx[0], (1, 0)).reshape(_NW, e_per_w)
    edge_pad = jnp.pad(edge_w, ((0, 0), (0, pad_w - e_per_w)))
    edge_blocks = edge_pad.reshape(_NW, pad_w // _CHUNK, _CHUNK)
    sj = _neighbor_states(S_flat, edge_blocks).reshape(K, N)

    Jt = jnp.transpose(J[0], (2, 3, 1, 0))
    h_cn = jnp.transpose(h[0], (1, 0))
    mij_kn = jnp.transpose(mask_ij[0], (1, 0))

    body = functools.partial(_tc_body, c=C, k=K)
    U, Ui = pl.pallas_call(
        body,
        grid=(C,),
        in_specs=[
            pl.BlockSpec((K, N), lambda i: (0, 0)),
            pl.BlockSpec((K, N), lambda i: (0, 0)),
            pl.BlockSpec((C, N), lambda i: (0, 0)),
            pl.BlockSpec((1, N), lambda i: (0, 0)),
            pl.BlockSpec((1, N), lambda i: (0, 0)),
            pl.BlockSpec((1, C, K, N), lambda i: (i, 0, 0, 0)),
        ],
        out_specs=[
            pl.BlockSpec((1, 1), lambda i: (0, 0)),
            pl.BlockSpec((C, N), lambda i: (0, 0)),
        ],
        out_shape=[
            jax.ShapeDtypeStruct((1, 1), jnp.float32),
            jax.ShapeDtypeStruct((C, N), jnp.float32),
        ],
        scratch_shapes=[pltpu.VMEM((C, K, N), jnp.float32)],
    )(sj, mij_kn, h_cn, mask_i, S, Jt)
    return (U.reshape(1), jnp.transpose(Ui, (1, 0))[None])
